# baseline (speedup 1.0000x reference)
import functools

import numpy as np
import jax
import jax.numpy as jnp
from jax import lax
from jax.experimental import pallas as pl
from jax.experimental.pallas import tpu as pltpu
from jax.experimental.pallas import tpu_sc as plsc

_MAJOR = np.array([30, 32, 33, 46, 47, 48, 50, 54, 76, 82, 84, 88, 90], dtype=np.int32)
_ACC = np.array([10, 11, 16, 20, 24, 35, 36, 53, 62, 63, 71, 73, 74, 77, 85, 93], dtype=np.int32)
_POS0 = np.concatenate([_MAJOR, _ACC]) - 1
_P = 99
_G = 22
_E = 128
_NPOS = 29
_NSEL = 32
_RW = 16
_RPP = 3

_NW = 32
_CB = 32
_PAIRS = _CB * _NSEL
_NREC = _PAIRS * _RPP
_SEG = 128
_NSEGC = _NREC // _SEG


def _tables(batch):
    posx = np.concatenate([_POS0, np.full(_NSEL - _NPOS, _POS0[0])]).astype(np.int64)
    q = np.arange(batch * _NSEL, dtype=np.int64)
    r = q >> 5
    w0 = r * (_P * _G) + posx[q & 31] * _G
    g0 = w0 >> 4
    idx = (g0[:, None] + np.arange(_RPP)[None, :]).reshape(-1)
    obase = (q % _PAIRS) * (_RW * _RPP) + (w0 & 15)
    return idx.astype(np.int32), obase.astype(np.int32)


_IDX, _OBASE = _tables(16384)


def _sc_mut(xw, idx_all, obase_all):
    nrecs = xw.shape[0]
    batch = nrecs * _RW // (_P * _G)
    bp = batch // _NW
    nchunks = bp // _CB
    mesh = plsc.VectorSubcoreMesh(core_axis_name="c", subcore_axis_name="s")

    @functools.partial(
        pl.kernel,
        mesh=mesh,
        out_type=jax.ShapeDtypeStruct((batch * _NSEL,), jnp.float32),
        compiler_params=pltpu.CompilerParams(
            needs_layout_passes=False,
            use_tc_tiling_on_sc=False,
        ),
        scratch_types=[
            pltpu.VMEM((_NREC,), jnp.int32),
            pltpu.VMEM((_NREC,), jnp.int32),
            pltpu.VMEM((_NREC, _RW), jnp.float32),
            pltpu.VMEM((_NREC, _RW), jnp.float32),
            pltpu.VMEM((_PAIRS,), jnp.int32),
            pltpu.VMEM((_PAIRS,), jnp.float32),
            pltpu.SemaphoreType.DMA,
            pltpu.SemaphoreType.DMA,
        ],
    )
    def k(xw_hbm, idx_hbm, obase_hbm, out_hbm, idx0, idx1, rec0, rec1,
          obase_v, mut_v, sem0, sem1):
        wid = lax.axis_index("s") * 2 + lax.axis_index("c")
        wbase = wid * bp
        idx_bufs = (idx0, idx1)
        rec_bufs = (rec0, rec1)
        sems = (sem0, sem1)
        iota = lax.iota(jnp.int32, 16)

        def start(c):
            s = c % 2
            qb = (wbase + c * _CB) * _NSEL
            pltpu.sync_copy(idx_hbm.at[pl.ds(qb * _RPP, _NREC)], idx_bufs[s])
            return [
                pltpu.async_copy(
                    xw_hbm.at[idx_bufs[s].at[pl.ds(sg * _SEG, _SEG)]],
                    rec_bufs[s].at[pl.ds(sg * _SEG, _SEG)], sems[s])
                for sg in range(_NSEGC)
            ]

        def compute(c):
            rec = rec_bufs[c % 2]
            qb = (wbase + c * _CB) * _NSEL
            pltpu.sync_copy(obase_hbm.at[pl.ds(qb, _PAIRS)], obase_v)

            def body(g, _):
                base = obase_v[pl.ds(g * 16, 16)]
                acc = plsc.load_gather(rec, [base >> 4, base & 15])
                for kk in range(1, _G - 1):
                    w = base + kk
                    acc = acc + plsc.load_gather(rec, [w >> 4, w & 15])
                mut_v[pl.ds(g * 16, 16)] = jnp.where(acc > 0.0, 1.0, 0.0)
                return _
            lax.fori_loop(0, _PAIRS // 16, body, 0)
            pltpu.sync_copy(mut_v, out_hbm.at[pl.ds(qb, _PAIRS)])

        handles = start(0)
        for c in range(nchunks):
            nxt = start(c + 1) if c + 1 < nchunks else None
            for h in handles:
                h.wait()
            compute(c)
            handles = nxt

    return k(xw, idx_all, obase_all).reshape(batch, _NSEL)


def _tc_body(mut_ref, mt_ref, at_ref, w_ref, b_ref, out_ref):
    pm = lax.dot_general(mt_ref[...], w_ref[:, :_E], (((1,), (1,)), ((), ())),
                         preferred_element_type=jnp.float32)
    pa = lax.dot_general(at_ref[...], w_ref[:, _E:], (((1,), (1,)), ((), ())),
                         preferred_element_type=jnp.float32)
    proj = jnp.concatenate(
        [pm, pa, jnp.zeros((_NSEL - _NPOS, _E), jnp.float32)], axis=0)
    out_ref[...] = lax.dot_general(mut_ref[...], proj, (((1,), (0,)), ((), ())),
                                   preferred_element_type=jnp.float32) + b_ref[...]


def kernel(x, major_table, accessory_table, W, b):
    batch, feat = x.shape
    xw = x.reshape(batch * feat // _RW, _RW)
    mut = _sc_mut(xw, jnp.asarray(_IDX), jnp.asarray(_OBASE))

    bb = 2048
    b2 = b.reshape(1, _E)
    return pl.pallas_call(
        _tc_body,
        grid=(batch // bb,),
        in_specs=[
            pl.BlockSpec((bb, _NSEL), lambda i: (i, 0)),
            pl.BlockSpec(major_table.shape, lambda i: (0, 0)),
            pl.BlockSpec(accessory_table.shape, lambda i: (0, 0)),
            pl.BlockSpec(W.shape, lambda i: (0, 0)),
            pl.BlockSpec((1, _E), lambda i: (0, 0)),
        ],
        out_specs=pl.BlockSpec((bb, _E), lambda i: (i, 0)),
        out_shape=jax.ShapeDtypeStruct((batch, _E), jnp.float32),
        compiler_params=pltpu.CompilerParams(
            dimension_semantics=("parallel",),
        ),
    )(mut, major_table, accessory_table, W, b2)


# speedup vs baseline: 1.1471x; 1.1471x over previous
import functools

import numpy as np
import jax
import jax.numpy as jnp
from jax import lax
from jax.experimental import pallas as pl
from jax.experimental.pallas import tpu as pltpu
from jax.experimental.pallas import tpu_sc as plsc

_MAJOR = np.array([30, 32, 33, 46, 47, 48, 50, 54, 76, 82, 84, 88, 90], dtype=np.int32)
_ACC = np.array([10, 11, 16, 20, 24, 35, 36, 53, 62, 63, 71, 73, 74, 77, 85, 93], dtype=np.int32)
_POS0 = np.concatenate([_MAJOR, _ACC]) - 1
_P = 99
_G = 22
_E = 128
_NPOS = 29
_NSEL = 32
_RW = 16
_RPP = 3

_NW = 32
_CB = 32
_PAIRS = _CB * _NSEL
_NREC = _PAIRS * _RPP
_SEG = 128
_NSEGC = _NREC // _SEG

_B = 16384
_S = 12288


def _tables(row0, nrows):
    posx = np.concatenate([_POS0, np.full(_NSEL - _NPOS, _POS0[0])]).astype(np.int64)
    q = np.arange(nrows * _NSEL, dtype=np.int64)
    r = row0 + (q >> 5)
    w0 = r * (_P * _G) + posx[q & 31] * _G
    g0 = w0 >> 4
    idx = (g0[:, None] + np.arange(_RPP)[None, :]).reshape(-1)
    obase = (q % _PAIRS) * (_RW * _RPP) + (w0 & 15)
    return idx.astype(np.int32), obase.astype(np.int32)


_IDX, _OBASE = _tables(_S, _B - _S)


def _sc_mut(xw, idx_all, obase_all, nrows):
    bp = nrows // _NW
    nchunks = bp // _CB
    mesh = plsc.VectorSubcoreMesh(core_axis_name="c", subcore_axis_name="s")

    @functools.partial(
        pl.kernel,
        mesh=mesh,
        out_type=jax.ShapeDtypeStruct((nrows * _NSEL,), jnp.float32),
        compiler_params=pltpu.CompilerParams(
            needs_layout_passes=False,
            use_tc_tiling_on_sc=False,
        ),
        scratch_types=[
            pltpu.VMEM((_NREC,), jnp.int32),
            pltpu.VMEM((_NREC,), jnp.int32),
            pltpu.VMEM((_NREC, _RW), jnp.float32),
            pltpu.VMEM((_NREC, _RW), jnp.float32),
            pltpu.VMEM((_PAIRS,), jnp.int32),
            pltpu.VMEM((_PAIRS,), jnp.float32),
            pltpu.SemaphoreType.DMA,
            pltpu.SemaphoreType.DMA,
        ],
    )
    def k(xw_hbm, idx_hbm, obase_hbm, out_hbm, idx0, idx1, rec0, rec1,
          obase_v, mut_v, sem0, sem1):
        wid = lax.axis_index("s") * 2 + lax.axis_index("c")
        wbase = wid * bp
        idx_bufs = (idx0, idx1)
        rec_bufs = (rec0, rec1)
        sems = (sem0, sem1)

        def start(c):
            s = c % 2
            qb = (wbase + c * _CB) * _NSEL
            pltpu.sync_copy(idx_hbm.at[pl.ds(qb * _RPP, _NREC)], idx_bufs[s])
            return [
                pltpu.async_copy(
                    xw_hbm.at[idx_bufs[s].at[pl.ds(sg * _SEG, _SEG)]],
                    rec_bufs[s].at[pl.ds(sg * _SEG, _SEG)], sems[s])
                for sg in range(_NSEGC)
            ]

        def compute(c):
            rec = rec_bufs[c % 2]
            qb = (wbase + c * _CB) * _NSEL
            pltpu.sync_copy(obase_hbm.at[pl.ds(qb, _PAIRS)], obase_v)

            def body(g, _):
                base = obase_v[pl.ds(g * 16, 16)]
                acc = plsc.load_gather(rec, [base >> 4, base & 15])
                for kk in range(1, _G - 1):
                    w = base + kk
                    acc = acc + plsc.load_gather(rec, [w >> 4, w & 15])
                mut_v[pl.ds(g * 16, 16)] = jnp.where(acc > 0.0, 1.0, 0.0)
                return _
            lax.fori_loop(0, _PAIRS // 16, body, 0)
            pltpu.sync_copy(mut_v, out_hbm.at[pl.ds(qb, _PAIRS)])

        handles = start(0)
        for c in range(nchunks):
            nxt = start(c + 1) if c + 1 < nchunks else None
            for h in handles:
                h.wait()
            compute(c)
            handles = nxt

    return k(xw, idx_all, obase_all).reshape(nrows, _NSEL)


def _sel_matrix():
    sel = np.zeros((_P * _G, _NSEL), np.float32)
    for j, pos in enumerate(_POS0):
        sel[_G * pos: _G * pos + 21, j] = 1.0
    return sel


_SEL = _sel_matrix()


def _proj(mt_ref, at_ref, w_ref):
    pm = lax.dot_general(mt_ref[...], w_ref[:, :_E], (((1,), (1,)), ((), ())),
                         preferred_element_type=jnp.float32)
    pa = lax.dot_general(at_ref[...], w_ref[:, _E:], (((1,), (1,)), ((), ())),
                         preferred_element_type=jnp.float32)
    return jnp.concatenate(
        [pm, pa, jnp.zeros((_NSEL - _NPOS, _E), jnp.float32)], axis=0)


def _tc_full_body(x_ref, sel_ref, mt_ref, at_ref, w_ref, b_ref, out_ref):
    xb = x_ref[...].astype(jnp.bfloat16)
    sums = lax.dot_general(xb, sel_ref[...], (((1,), (0,)), ((), ())),
                           preferred_element_type=jnp.float32)
    mut = (sums > 0).astype(jnp.float32)
    out_ref[...] = lax.dot_general(mut, _proj(mt_ref, at_ref, w_ref),
                                   (((1,), (0,)), ((), ())),
                                   preferred_element_type=jnp.float32) + b_ref[...]


def _tc_mut_body(mut_ref, mt_ref, at_ref, w_ref, b_ref, out_ref):
    out_ref[...] = lax.dot_general(mut_ref[...], _proj(mt_ref, at_ref, w_ref),
                                   (((1,), (0,)), ((), ())),
                                   preferred_element_type=jnp.float32) + b_ref[...]


def kernel(x, major_table, accessory_table, W, b):
    batch, feat = x.shape
    xw = x.reshape(batch * feat // _RW, _RW)
    b2 = b.reshape(1, _E)
    sel = jnp.asarray(_SEL, dtype=jnp.bfloat16)

    mut = _sc_mut(xw, jnp.asarray(_IDX), jnp.asarray(_OBASE), batch - _S)

    bb = 512
    out_tc = pl.pallas_call(
        _tc_full_body,
        grid=(_S // bb,),
        in_specs=[
            pl.BlockSpec((bb, feat), lambda i: (i, 0)),
            pl.BlockSpec((feat, _NSEL), lambda i: (0, 0)),
            pl.BlockSpec(major_table.shape, lambda i: (0, 0)),
            pl.BlockSpec(accessory_table.shape, lambda i: (0, 0)),
            pl.BlockSpec(W.shape, lambda i: (0, 0)),
            pl.BlockSpec((1, _E), lambda i: (0, 0)),
        ],
        out_specs=pl.BlockSpec((bb, _E), lambda i: (i, 0)),
        out_shape=jax.ShapeDtypeStruct((_S, _E), jnp.float32),
        compiler_params=pltpu.CompilerParams(
            dimension_semantics=("parallel",),
        ),
    )(x, sel, major_table, accessory_table, W, b2)

    bb2 = 2048
    out_sc = pl.pallas_call(
        _tc_mut_body,
        grid=((batch - _S) // bb2,),
        in_specs=[
            pl.BlockSpec((bb2, _NSEL), lambda i: (i, 0)),
            pl.BlockSpec(major_table.shape, lambda i: (0, 0)),
            pl.BlockSpec(accessory_table.shape, lambda i: (0, 0)),
            pl.BlockSpec(W.shape, lambda i: (0, 0)),
            pl.BlockSpec((1, _E), lambda i: (0, 0)),
        ],
        out_specs=pl.BlockSpec((bb2, _E), lambda i: (i, 0)),
        out_shape=jax.ShapeDtypeStruct((batch - _S, _E), jnp.float32),
        compiler_params=pltpu.CompilerParams(
            dimension_semantics=("parallel",),
        ),
    )(mut, major_table, accessory_table, W, b2)

    return jnp.concatenate([out_tc, out_sc], axis=0)


# speedup vs baseline: 2.8136x; 2.4528x over previous
import numpy as np
import jax
import jax.numpy as jnp
from jax import lax
from jax.experimental import pallas as pl
from jax.experimental.pallas import tpu as pltpu

_MAJOR = np.array([30, 32, 33, 46, 47, 48, 50, 54, 76, 82, 84, 88, 90], dtype=np.int32)
_ACC = np.array([10, 11, 16, 20, 24, 35, 36, 53, 62, 63, 71, 73, 74, 77, 85, 93], dtype=np.int32)
_P = 99
_E = 128
_NPOS = len(_MAJOR) + len(_ACC)
_NSEL = 32


def _sel_matrix() -> np.ndarray:
    sel = np.zeros((_P * 22, _NSEL), np.float32)
    for j, pos in enumerate(np.concatenate([_MAJOR, _ACC])):
        q = int(pos) - 1
        sel[22 * q: 22 * q + 21, j] = 1.0
    return sel


_SEL = _sel_matrix()


def _body(x_ref, sel_ref, mt_ref, at_ref, w_ref, b_ref, out_ref):
    xb = x_ref[...].astype(jnp.bfloat16)
    sums = lax.dot_general(xb, sel_ref[...], (((1,), (0,)), ((), ())),
                           preferred_element_type=jnp.float32)
    mut = (sums > 0).astype(jnp.float32)
    pm = lax.dot_general(mt_ref[...], w_ref[:, :_E], (((1,), (1,)), ((), ())),
                         preferred_element_type=jnp.float32)
    pa = lax.dot_general(at_ref[...], w_ref[:, _E:], (((1,), (1,)), ((), ())),
                         preferred_element_type=jnp.float32)
    proj = jnp.concatenate([pm, pa, jnp.zeros((_NSEL - _NPOS, _E), jnp.float32)], axis=0)
    out_ref[...] = lax.dot_general(mut, proj, (((1,), (0,)), ((), ())),
                                   preferred_element_type=jnp.float32) + b_ref[...]


def kernel(x, major_table, accessory_table, W, b):
    batch, feat = x.shape
    bb = 512
    grid = (batch // bb,)
    sel = jnp.asarray(_SEL, dtype=jnp.bfloat16)
    b2 = b.reshape(1, _E)
    return pl.pallas_call(
        _body,
        grid=grid,
        in_specs=[
            pl.BlockSpec((bb, feat), lambda i: (i, 0)),
            pl.BlockSpec((feat, _NSEL), lambda i: (0, 0)),
            pl.BlockSpec(major_table.shape, lambda i: (0, 0)),
            pl.BlockSpec(accessory_table.shape, lambda i: (0, 0)),
            pl.BlockSpec(W.shape, lambda i: (0, 0)),
            pl.BlockSpec((1, _E), lambda i: (0, 0)),
        ],
        out_specs=pl.BlockSpec((bb, _E), lambda i: (i, 0)),
        out_shape=jax.ShapeDtypeStruct((batch, _E), jnp.float32),
        compiler_params=pltpu.CompilerParams(
            dimension_semantics=("parallel",),
        ),
    )(x, sel, major_table, accessory_table, W, b2)
